# baseline (speedup 1.0000x reference)
import functools

import jax
import jax.numpy as jnp
from jax.experimental import pallas as pl
from jax.experimental.pallas import tpu as pltpu

_IOU_T = 0.7
_MAX_OUT = 256
_B = 128
_W = 1024


def _overlap_mask(x1c, y1c, x2c, y2c, x1r, y1r, x2r, y2r):
    area_c = (x2c - x1c + 1.0) * (y2c - y1c + 1.0)
    area_r = (x2r - x1r + 1.0) * (y2r - y1r + 1.0)
    xx1 = jnp.maximum(x1c, x1r)
    yy1 = jnp.maximum(y1c, y1r)
    xx2 = jnp.minimum(x2c, x2r)
    yy2 = jnp.minimum(y2c, y2r)
    w = jnp.maximum(0.0, xx2 - xx1 + 1.0)
    h = jnp.maximum(0.0, yy2 - yy1 + 1.0)
    inter = w * h
    denom = jnp.minimum(area_c, area_r)
    o = inter / denom
    return o > _IOU_T


def _dotg(a, b, dims):
    return jax.lax.dot_general(
        a, b, (dims, ((), ())),
        precision=jax.lax.Precision.HIGHEST,
        preferred_element_type=jnp.float32)


def _nms_body(n, nblk, np_, npad, dataU_ref, dataUrm_ref, ord_ref,
              out_ref, misc_ref, data_ref, dataT_ref, keep_ref):
    data_ref[:, np_:npad] = jnp.zeros((9, npad - np_), jnp.float32)
    keep_ref[...] = jnp.where(
        jax.lax.broadcasted_iota(jnp.int32, (1, npad), 1) < n, 1.0, 0.0)
    dataU = dataU_ref[...]
    dataUrm = dataUrm_ref[...]
    csub = jax.lax.broadcasted_iota(jnp.int32, (npad, _B), 0)

    def gather_step(sb, _):
        base = sb * _B
        ordb = ord_ref[0:1, pl.ds(base, _B)]
        onehotT = jnp.where(
            csub.astype(jnp.float32) == ordb, 1.0, 0.0)
        rowform = _dotg(dataU, onehotT, (((1,), (0,))))
        blkform = _dotg(onehotT, dataUrm, (((0,), (0,))))
        data_ref[:, pl.ds(base, _B)] = rowform
        dataT_ref[pl.ds(base, _B), :] = blkform
        return 0

    jax.lax.fori_loop(0, nblk, gather_step, 0)

    def block_step(b, _):
        base = b * _B
        blk = dataT_ref[pl.ds(base, _B), :]
        x1c = blk[:, 0:1]
        y1c = blk[:, 1:2]
        x2c = blk[:, 2:3]
        y2c = blk[:, 3:4]
        irow = jax.lax.broadcasted_iota(jnp.int32, (_B, _W), 0) + base

        def chunk_mask(start):
            x1r = data_ref[0:1, pl.ds(start, _W)]
            y1r = data_ref[1:2, pl.ds(start, _W)]
            x2r = data_ref[2:3, pl.ds(start, _W)]
            y2r = data_ref[3:4, pl.ds(start, _W)]
            om = _overlap_mask(x1c, y1c, x2c, y2c, x1r, y1r, x2r, y2r)
            jcol = jax.lax.broadcasted_iota(jnp.int32, (_B, _W), 1) + start
            return jnp.where(om & (jcol > irow), 1.0, 0.0)

        def apply_chunk(kfin, start, mf):
            supp = _dotg(kfin, mf, ((1,), (0,)))
            cur = keep_ref[0:1, pl.ds(start, _W)]
            keep_ref[0:1, pl.ds(start, _W)] = jnp.where(
                supp > 0.5, 0.0, cur)

        mf0 = chunk_mask(base)
        mintra = mf0[:, 0:_B]
        kinit = keep_ref[0:1, pl.ds(base, _B)]

        def cond(c):
            kp, k = c
            return jnp.any(kp != k)

        def body(c):
            _, k = c
            supp = _dotg(k, mintra, ((1,), (0,)))
            knew = jnp.where(supp > 0.5, 0.0, kinit)
            return (k, knew)

        _, kfin = jax.lax.while_loop(cond, body, (kinit - 1.0, kinit))

        apply_chunk(kfin, base, mf0)
        nc = (np_ - base + _W - 1) // _W

        def chunk_step(c, _):
            start = base + c * _W
            apply_chunk(kfin, start, chunk_mask(start))
            return 0

        jax.lax.fori_loop(1, nc, chunk_step, 0)
        return 0

    jax.lax.fori_loop(0, nblk, block_step, 0)

    keep = keep_ref[0:1, 0:np_]
    data = data_ref[...]
    count = jnp.sum(keep)
    c = keep
    s = 1
    while s < np_:
        shifted = jnp.concatenate(
            [jnp.zeros((1, s), jnp.float32), c[:, : np_ - s]], axis=1)
        c = c + shifted
        s *= 2
    sval = (jax.lax.broadcasted_iota(jnp.int32, (_MAX_OUT, 1), 0) + 1
            ).astype(jnp.float32)
    onehot = jnp.where((c == sval) & (keep > 0.5), 1.0, 0.0)

    def gath(row):
        return jnp.sum(onehot * row[:, 0:np_], axis=1, keepdims=True)

    vc = jnp.where(
        jax.lax.broadcasted_iota(jnp.int32, (_MAX_OUT, 1), 0).astype(
            jnp.float32) < count,
        1.0, 0.0)
    ordrow = ord_ref[...]
    pk = gath(ordrow)
    pk = jnp.where(vc > 0.5, pk, ordrow[0:1, 0:1])

    out_ref[:, 0:1] = gath(data[4:5, :])
    out_ref[:, 1:2] = gath(data[5:6, :])
    out_ref[:, 2:3] = gath(data[6:7, :])
    out_ref[:, 3:4] = gath(data[7:8, :])
    out_ref[:, 4:5] = gath(data[8:9, :])
    out_ref[:, 5:8] = jnp.zeros((_MAX_OUT, 3), jnp.float32)
    misc_ref[:, 0:1] = pk
    misc_ref[:, 1:2] = vc
    misc_ref[:, 2:8] = jnp.zeros((_MAX_OUT, 6), jnp.float32)


def kernel(boxes, scores, idxs):
    n = boxes.shape[0]
    nblk = (n + _B - 1) // _B
    np_ = nblk * _B
    npad = np_ + _W

    max_coordinate = boxes.max()
    offsets = idxs.astype(boxes.dtype) * (max_coordinate + 1.0)
    boxes_for_nms = boxes + offsets[:, None]
    order = jnp.argsort(-scores)
    orderf = order.astype(jnp.float32)

    pad = npad - n
    cols = [boxes_for_nms[:, 0], boxes_for_nms[:, 1],
            boxes_for_nms[:, 2], boxes_for_nms[:, 3],
            boxes[:, 0], boxes[:, 1], boxes[:, 2], boxes[:, 3],
            scores]
    padded = [jnp.pad(cc, (0, pad)) for cc in cols]
    dataU = jnp.stack(padded, axis=0)
    dataUrm = jnp.stack(padded, axis=1)
    ordrow = jnp.pad(orderf, (0, pad))[None, :]

    out8, misc = pl.pallas_call(
        functools.partial(_nms_body, n, nblk, np_, npad),
        out_shape=[
            jax.ShapeDtypeStruct((_MAX_OUT, 8), jnp.float32),
            jax.ShapeDtypeStruct((_MAX_OUT, 8), jnp.float32),
        ],
        scratch_shapes=[
            pltpu.VMEM((9, npad), jnp.float32),
            pltpu.VMEM((npad, 9), jnp.float32),
            pltpu.VMEM((1, npad), jnp.float32),
        ],
    )(dataU, dataUrm, ordrow)

    out = out8[:, :5]
    picks = misc[:, 0].astype(jnp.int32)
    valid = misc[:, 1] > 0.5
    return out, picks, valid


# speedup vs baseline: 1.9141x; 1.9141x over previous
import functools

import jax
import jax.numpy as jnp
from jax.experimental import pallas as pl
from jax.experimental.pallas import tpu as pltpu

_IOU_T = 0.7
_MAX_OUT = 256
_B = 128
_W = 1024


def _overlap_mask(x1c, y1c, x2c, y2c, x1r, y1r, x2r, y2r):
    area_c = (x2c - x1c + 1.0) * (y2c - y1c + 1.0)
    area_r = (x2r - x1r + 1.0) * (y2r - y1r + 1.0)
    xx1 = jnp.maximum(x1c, x1r)
    yy1 = jnp.maximum(y1c, y1r)
    xx2 = jnp.minimum(x2c, x2r)
    yy2 = jnp.minimum(y2c, y2r)
    w = jnp.maximum(0.0, xx2 - xx1 + 1.0)
    h = jnp.maximum(0.0, yy2 - yy1 + 1.0)
    inter = w * h
    denom = jnp.minimum(area_c, area_r)
    o = inter / denom
    return o > _IOU_T


def _nms_body(n, nblk, np_, npad, data_ref, dataT_ref, dataU_ref, ord_ref,
              out_ref, misc_ref, keep_ref):
    keep_ref[...] = jnp.where(
        jax.lax.broadcasted_iota(jnp.int32, (1, npad), 1) < n, 1.0, 0.0)

    def block_step(b, _):
        base = b * _B
        blk = dataT_ref[pl.ds(base, _B), :]
        x1c = blk[:, 0:1]
        y1c = blk[:, 1:2]
        x2c = blk[:, 2:3]
        y2c = blk[:, 3:4]
        irow = jax.lax.broadcasted_iota(jnp.int32, (_B, _W), 0) + base

        def chunk_mask(start):
            x1r = data_ref[0:1, pl.ds(start, _W)]
            y1r = data_ref[1:2, pl.ds(start, _W)]
            x2r = data_ref[2:3, pl.ds(start, _W)]
            y2r = data_ref[3:4, pl.ds(start, _W)]
            om = _overlap_mask(x1c, y1c, x2c, y2c, x1r, y1r, x2r, y2r)
            jcol = jax.lax.broadcasted_iota(jnp.int32, (_B, _W), 1) + start
            return jnp.where(om & (jcol > irow), 1.0, 0.0)

        def apply_chunk(kfin, start, mf):
            supp = jax.lax.dot_general(
                kfin, mf, (((1,), (0,)), ((), ())),
                precision=jax.lax.Precision.HIGHEST,
                preferred_element_type=jnp.float32)
            cur = keep_ref[0:1, pl.ds(start, _W)]
            keep_ref[0:1, pl.ds(start, _W)] = jnp.where(
                supp > 0.5, 0.0, cur)

        mf0 = chunk_mask(base)
        mintra = mf0[:, 0:_B]
        kinit = keep_ref[0:1, pl.ds(base, _B)]

        def cond(c):
            kp, k = c
            return jnp.any(kp != k)

        def body(c):
            _, k = c
            supp = jax.lax.dot_general(
                k, mintra, (((1,), (0,)), ((), ())),
                precision=jax.lax.Precision.HIGHEST,
                preferred_element_type=jnp.float32)
            knew = jnp.where(supp > 0.5, 0.0, kinit)
            return (k, knew)

        _, kfin = jax.lax.while_loop(cond, body, (kinit - 1.0, kinit))

        apply_chunk(kfin, base, mf0)
        nc = (np_ - base + _W - 1) // _W

        def chunk_step(c, _):
            start = base + c * _W
            apply_chunk(kfin, start, chunk_mask(start))
            return 0

        jax.lax.fori_loop(1, nc, chunk_step, 0)
        return 0

    jax.lax.fori_loop(0, nblk, block_step, 0)

    keep = keep_ref[0:1, 0:np_]
    count = jnp.sum(keep)
    c = keep
    s = 1
    while s < np_:
        shifted = jnp.concatenate(
            [jnp.zeros((1, s), jnp.float32), c[:, : np_ - s]], axis=1)
        c = c + shifted
        s *= 2
    sval = (jax.lax.broadcasted_iota(jnp.int32, (_MAX_OUT, 1), 0) + 1
            ).astype(jnp.float32)
    onehot = jnp.where((c == sval) & (keep > 0.5), 1.0, 0.0)

    vc = jnp.where(
        jax.lax.broadcasted_iota(jnp.int32, (_MAX_OUT, 1), 0).astype(
            jnp.float32) < count,
        1.0, 0.0)
    ordrow = ord_ref[...]
    pk = jnp.sum(onehot * ordrow[:, 0:np_], axis=1, keepdims=True)
    pk = jnp.where(vc > 0.5, pk, ordrow[0:1, 0:1])

    jorig = jax.lax.broadcasted_iota(jnp.int32, (_MAX_OUT, np_), 1).astype(
        jnp.float32)
    onehot2 = jnp.where((jorig == pk) & (vc > 0.5), 1.0, 0.0)
    dataU = dataU_ref[...]

    def gath2(row):
        return jnp.sum(onehot2 * row[:, 0:np_], axis=1, keepdims=True)

    out_ref[:, 0:1] = gath2(dataU[0:1, :])
    out_ref[:, 1:2] = gath2(dataU[1:2, :])
    out_ref[:, 2:3] = gath2(dataU[2:3, :])
    out_ref[:, 3:4] = gath2(dataU[3:4, :])
    out_ref[:, 4:5] = gath2(dataU[4:5, :])
    out_ref[:, 5:8] = jnp.zeros((_MAX_OUT, 3), jnp.float32)
    misc_ref[:, 0:1] = pk
    misc_ref[:, 1:2] = vc
    misc_ref[:, 2:8] = jnp.zeros((_MAX_OUT, 6), jnp.float32)


def kernel(boxes, scores, idxs):
    n = boxes.shape[0]
    nblk = (n + _B - 1) // _B
    np_ = nblk * _B
    npad = np_ + _W

    max_coordinate = boxes.max()
    offsets = idxs.astype(boxes.dtype) * (max_coordinate + 1.0)
    boxes_for_nms = boxes + offsets[:, None]
    order = jnp.argsort(-scores)
    bo = boxes_for_nms[order]
    orderf = order.astype(jnp.float32)

    pad = npad - n
    dataT = jnp.pad(bo, ((0, pad), (0, 0)))
    data = dataT.T
    dataU = jnp.stack(
        [jnp.pad(boxes[:, 0], (0, pad)), jnp.pad(boxes[:, 1], (0, pad)),
         jnp.pad(boxes[:, 2], (0, pad)), jnp.pad(boxes[:, 3], (0, pad)),
         jnp.pad(scores, (0, pad))], axis=0)
    ordrow = jnp.pad(orderf, (0, pad))[None, :]

    out8, misc = pl.pallas_call(
        functools.partial(_nms_body, n, nblk, np_, npad),
        out_shape=[
            jax.ShapeDtypeStruct((_MAX_OUT, 8), jnp.float32),
            jax.ShapeDtypeStruct((_MAX_OUT, 8), jnp.float32),
        ],
        scratch_shapes=[pltpu.VMEM((1, npad), jnp.float32)],
    )(data, dataT, dataU, ordrow)

    out = out8[:, :5]
    picks = misc[:, 0].astype(jnp.int32)
    valid = misc[:, 1] > 0.5
    return out, picks, valid


# speedup vs baseline: 1.9872x; 1.0382x over previous
import functools

import jax
import jax.numpy as jnp
from jax.experimental import pallas as pl
from jax.experimental.pallas import tpu as pltpu

_IOU_T = 0.7
_MAX_OUT = 256
_B = 128
_W = 1024


def _overlap_mask(x1c, y1c, x2c, y2c, x1r, y1r, x2r, y2r):
    area_c = (x2c - x1c + 1.0) * (y2c - y1c + 1.0)
    area_r = (x2r - x1r + 1.0) * (y2r - y1r + 1.0)
    xx1 = jnp.maximum(x1c, x1r)
    yy1 = jnp.maximum(y1c, y1r)
    xx2 = jnp.minimum(x2c, x2r)
    yy2 = jnp.minimum(y2c, y2r)
    w = jnp.maximum(0.0, xx2 - xx1 + 1.0)
    h = jnp.maximum(0.0, yy2 - yy1 + 1.0)
    inter = w * h
    denom = jnp.minimum(area_c, area_r)
    o = inter / denom
    return o > _IOU_T


def _nms_body(n, nblk, np_, npad, data_ref, dataT_ref, dataU_ref, ord_ref,
              out_ref, misc_ref, keep_ref):
    keep_ref[...] = jnp.where(
        jax.lax.broadcasted_iota(jnp.int32, (1, npad), 1) < n, 1.0, 0.0)
    tri = (jax.lax.broadcasted_iota(jnp.int32, (_B, _W), 1) >
           jax.lax.broadcasted_iota(jnp.int32, (_B, _W), 0))

    def block_step(b, _):
        base = b * _B
        blk = dataT_ref[pl.ds(base, _B), :]
        x1c = blk[:, 0:1]
        y1c = blk[:, 1:2]
        x2c = blk[:, 2:3]
        y2c = blk[:, 3:4]

        def chunk_mask(start, mask):
            x1r = data_ref[0:1, pl.ds(start, _W)]
            y1r = data_ref[1:2, pl.ds(start, _W)]
            x2r = data_ref[2:3, pl.ds(start, _W)]
            y2r = data_ref[3:4, pl.ds(start, _W)]
            om = _overlap_mask(x1c, y1c, x2c, y2c, x1r, y1r, x2r, y2r)
            if mask is not None:
                om = om & mask
            return jnp.where(om, 1.0, 0.0)

        def apply_chunk(kfin, start, mf):
            supp = jax.lax.dot_general(
                kfin, mf, (((1,), (0,)), ((), ())),
                precision=jax.lax.Precision.HIGHEST,
                preferred_element_type=jnp.float32)
            cur = keep_ref[0:1, pl.ds(start, _W)]
            keep_ref[0:1, pl.ds(start, _W)] = jnp.where(
                supp > 0.5, 0.0, cur)

        mf0 = chunk_mask(base, tri)
        mintra = mf0[:, 0:_B]
        kinit = keep_ref[0:1, pl.ds(base, _B)]

        def cond(c):
            kp, k = c
            return jnp.any(kp != k)

        def body(c):
            _, k = c
            supp = jax.lax.dot_general(
                k, mintra, (((1,), (0,)), ((), ())),
                precision=jax.lax.Precision.HIGHEST,
                preferred_element_type=jnp.float32)
            knew = jnp.where(supp > 0.5, 0.0, kinit)
            return (k, knew)

        _, kfin = jax.lax.while_loop(cond, body, (kinit - 1.0, kinit))

        apply_chunk(kfin, base, mf0)
        nc = (np_ - base + _W - 1) // _W

        def chunk_step(c, _):
            start = base + c * _W
            apply_chunk(kfin, start, chunk_mask(start, None))
            return 0

        jax.lax.fori_loop(1, nc, chunk_step, 0)
        return 0

    jax.lax.fori_loop(0, nblk, block_step, 0)

    keep = keep_ref[0:1, 0:np_]
    count = jnp.sum(keep)
    c = keep
    s = 1
    while s < np_:
        shifted = jnp.concatenate(
            [jnp.zeros((1, s), jnp.float32), c[:, : np_ - s]], axis=1)
        c = c + shifted
        s *= 2
    sval = (jax.lax.broadcasted_iota(jnp.int32, (_MAX_OUT, 1), 0) + 1
            ).astype(jnp.float32)
    onehot = jnp.where((c == sval) & (keep > 0.5), 1.0, 0.0)

    vc = jnp.where(
        jax.lax.broadcasted_iota(jnp.int32, (_MAX_OUT, 1), 0).astype(
            jnp.float32) < count,
        1.0, 0.0)
    ordrow = ord_ref[...]
    pk = jnp.sum(onehot * ordrow[:, 0:np_], axis=1, keepdims=True)
    pk = jnp.where(vc > 0.5, pk, ordrow[0:1, 0:1])

    jorig = jax.lax.broadcasted_iota(jnp.int32, (_MAX_OUT, np_), 1).astype(
        jnp.float32)
    onehot2 = jnp.where((jorig == pk) & (vc > 0.5), 1.0, 0.0)
    dataU = dataU_ref[...]

    def gath2(row):
        return jnp.sum(onehot2 * row[:, 0:np_], axis=1, keepdims=True)

    out_ref[:, 0:1] = gath2(dataU[0:1, :])
    out_ref[:, 1:2] = gath2(dataU[1:2, :])
    out_ref[:, 2:3] = gath2(dataU[2:3, :])
    out_ref[:, 3:4] = gath2(dataU[3:4, :])
    out_ref[:, 4:5] = gath2(dataU[4:5, :])
    out_ref[:, 5:8] = jnp.zeros((_MAX_OUT, 3), jnp.float32)
    misc_ref[:, 0:1] = pk
    misc_ref[:, 1:2] = vc
    misc_ref[:, 2:8] = jnp.zeros((_MAX_OUT, 6), jnp.float32)


def kernel(boxes, scores, idxs):
    n = boxes.shape[0]
    nblk = (n + _B - 1) // _B
    np_ = nblk * _B
    npad = np_ + _W

    max_coordinate = boxes.max()
    offsets = idxs.astype(boxes.dtype) * (max_coordinate + 1.0)
    boxes_for_nms = boxes + offsets[:, None]
    order = jnp.argsort(-scores)
    bo = boxes_for_nms[order]
    orderf = order.astype(jnp.float32)

    pad = npad - n
    dataT = jnp.pad(bo, ((0, pad), (0, 0)))
    data = dataT.T
    dataU = jnp.stack(
        [jnp.pad(boxes[:, 0], (0, pad)), jnp.pad(boxes[:, 1], (0, pad)),
         jnp.pad(boxes[:, 2], (0, pad)), jnp.pad(boxes[:, 3], (0, pad)),
         jnp.pad(scores, (0, pad))], axis=0)
    ordrow = jnp.pad(orderf, (0, pad))[None, :]

    out8, misc = pl.pallas_call(
        functools.partial(_nms_body, n, nblk, np_, npad),
        out_shape=[
            jax.ShapeDtypeStruct((_MAX_OUT, 8), jnp.float32),
            jax.ShapeDtypeStruct((_MAX_OUT, 8), jnp.float32),
        ],
        scratch_shapes=[pltpu.VMEM((1, npad), jnp.float32)],
    )(data, dataT, dataU, ordrow)

    out = out8[:, :5]
    picks = misc[:, 0].astype(jnp.int32)
    valid = misc[:, 1] > 0.5
    return out, picks, valid
